# baseline (speedup 1.0000x reference)
import functools

import jax
import jax.numpy as jnp
from jax import lax
from jax.experimental import pallas as pl
from jax.experimental.pallas import tpu as pltpu
from jax.experimental.pallas import tpu_sc as plsc

D_MODEL = 768
NUM_CORES = 2
NUM_SUBCORES = 16
NW = NUM_CORES * NUM_SUBCORES
LANES = 16
GROUPS = D_MODEL // LANES

_mesh = plsc.VectorSubcoreMesh(core_axis_name="c", subcore_axis_name="s")


@functools.cache
def _build(n_seq: int, seq_len: int, rpc: int, nbuf: int, prefetch: int):
    ppw = seq_len // NW
    cpc = ppw // rpc
    n_chunks = n_seq * cpc
    total = n_seq * seq_len
    assert nbuf % cpc == 0 and n_chunks % nbuf == 0

    @functools.partial(
        pl.kernel,
        mesh=_mesh,
        out_type=jax.ShapeDtypeStruct((total, D_MODEL), jnp.float32),
        scratch_types=[
            pltpu.VMEM((n_seq * ppw,), jnp.int32),
            pltpu.VMEM((ppw, D_MODEL), jnp.float32),
        ]
        + [pltpu.VMEM((rpc, D_MODEL), jnp.float32) for _ in range(nbuf)]
        + [pltpu.SemaphoreType.DMA for _ in range(2 * nbuf)],
    )
    def emb(x_hbm, tok_hbm, pos_hbm, out_hbm, idx_all, pos_v, *bufs_sems):
        rows = bufs_sems[:nbuf]
        gsem = bufs_sems[nbuf:2 * nbuf]
        osem = bufs_sems[2 * nbuf:]
        wid = lax.axis_index("s") * NUM_CORES + lax.axis_index("c")
        p_lo = pl.multiple_of(wid * ppw, ppw)

        def idx_body(s, _):
            pltpu.async_copy(
                x_hbm.at[pl.ds(s * seq_len + p_lo, ppw)],
                idx_all.at[pl.ds(s * ppw, ppw)],
                gsem[0],
            )
            return 0

        lax.fori_loop(0, n_seq, idx_body, 0)
        pltpu.sync_copy(pos_hbm.at[pl.ds(p_lo, ppw)], pos_v)
        pltpu.make_async_copy(x_hbm.at[pl.ds(0, n_seq * ppw)],
                              idx_all, gsem[0]).wait()

        def start_gather(b, k):
            pltpu.async_copy(
                tok_hbm.at[idx_all.at[pl.ds(k * rpc, rpc)]], rows[b], gsem[b]
            )

        def wait_gather(b):
            pltpu.make_async_copy(
                tok_hbm.at[idx_all.at[pl.ds(0, rpc)]], rows[b], gsem[b]
            ).wait()

        def start_out(b, k):
            s = k // cpc
            h_off = (k % cpc) * rpc if isinstance(k, int) else lax.rem(k, cpc) * rpc
            pltpu.async_copy(
                rows[b], out_hbm.at[pl.ds(s * seq_len + p_lo + h_off, rpc)], osem[b]
            )

        def wait_out(b):
            pltpu.make_async_copy(rows[b], out_hbm.at[pl.ds(0, rpc)], osem[b]).wait()

        for k0 in range(prefetch):
            start_gather(k0, k0)

        def body(j, _):
            for b in range(nbuf):
                k = nbuf * j + b
                h_base = (b % cpc) * rpc
                wait_gather(b)

                def row_body(r, _):
                    for g in range(GROUPS):
                        sl = pl.ds(g * LANES, LANES)
                        rows[b][r, sl] = rows[b][r, sl] + pos_v[h_base + r, sl]
                    return 0

                lax.fori_loop(0, rpc, row_body, 0)

                s = (nbuf // cpc) * j + b // cpc
                pltpu.async_copy(
                    rows[b],
                    out_hbm.at[pl.ds(s * seq_len + p_lo + h_base, rpc)],
                    osem[b],
                )

                kn = k + prefetch
                bn = (b + prefetch) % nbuf

                @pl.when(kn < n_chunks)
                def _prefetch():
                    @pl.when(k >= nbuf - prefetch)
                    def _drain():
                        wait_out(bn)

                    start_gather(bn, kn)

            return 0

        lax.fori_loop(0, n_chunks // nbuf, body, 0)

        for b in range(nbuf):
            wait_out(b)

    return emb


def kernel(x, tok_table, pos_table):
    B, T = x.shape
    emb = _build(B, T, 16, 8, 4)
    out = emb(x.reshape(B * T).astype(jnp.int32), tok_table, pos_table)
    return out.reshape(B, T, D_MODEL)


# speedup vs baseline: 1.4676x; 1.4676x over previous
import functools

import jax
import jax.numpy as jnp
from jax import lax
from jax.experimental import pallas as pl
from jax.experimental.pallas import tpu as pltpu
from jax.experimental.pallas import tpu_sc as plsc

D_MODEL = 768
NUM_CORES = 2
NUM_SUBCORES = 16
NW = NUM_CORES * NUM_SUBCORES
LANES = 16
GROUPS = D_MODEL // LANES

_mesh = plsc.VectorSubcoreMesh(core_axis_name="c", subcore_axis_name="s")


@functools.cache
def _build(n_seq: int, seq_len: int, rpc: int, nbuf: int, prefetch: int):
    ppw = seq_len // NW
    cpc = ppw // rpc
    n_chunks = n_seq * cpc
    total = n_seq * seq_len
    assert nbuf % cpc == 0 and n_chunks % nbuf == 0

    @functools.partial(
        pl.kernel,
        mesh=_mesh,
        out_type=jax.ShapeDtypeStruct((total, D_MODEL), jnp.float32),
        scratch_types=[
            pltpu.VMEM((n_seq * ppw,), jnp.int32),
            pltpu.VMEM((ppw, D_MODEL), jnp.float32),
        ]
        + [pltpu.VMEM((rpc, D_MODEL), jnp.float32) for _ in range(nbuf)]
        + [pltpu.SemaphoreType.DMA for _ in range(2 * nbuf)],
    )
    def emb(x_hbm, tok_hbm, pos_hbm, out_hbm, idx_all, pos_v, *bufs_sems):
        rows = bufs_sems[:nbuf]
        gsem = bufs_sems[nbuf:2 * nbuf]
        osem = bufs_sems[2 * nbuf:]
        wid = lax.axis_index("s") * NUM_CORES + lax.axis_index("c")
        p_lo = pl.multiple_of(wid * ppw, ppw)

        def idx_body(s, _):
            pltpu.async_copy(
                x_hbm.at[pl.ds(s * seq_len + p_lo, ppw)],
                idx_all.at[pl.ds(s * ppw, ppw)],
                gsem[0],
            )
            return 0

        lax.fori_loop(0, n_seq, idx_body, 0)
        pltpu.sync_copy(pos_hbm.at[pl.ds(p_lo, ppw)], pos_v)
        pltpu.make_async_copy(x_hbm.at[pl.ds(0, n_seq * ppw)],
                              idx_all, gsem[0]).wait()

        def start_gather(b, k):
            pltpu.async_copy(
                tok_hbm.at[idx_all.at[pl.ds(k * rpc, rpc)]], rows[b], gsem[b]
            )

        def wait_gather(b):
            pltpu.make_async_copy(
                tok_hbm.at[idx_all.at[pl.ds(0, rpc)]], rows[b], gsem[b]
            ).wait()

        def start_out(b, k):
            s = k // cpc
            h_off = (k % cpc) * rpc if isinstance(k, int) else lax.rem(k, cpc) * rpc
            pltpu.async_copy(
                rows[b], out_hbm.at[pl.ds(s * seq_len + p_lo + h_off, rpc)], osem[b]
            )

        def wait_out(b):
            pltpu.make_async_copy(rows[b], out_hbm.at[pl.ds(0, rpc)], osem[b]).wait()

        for k0 in range(prefetch):
            start_gather(k0, k0)

        def body(j, _):
            for b in range(nbuf):
                k = nbuf * j + b
                h_base = (b % cpc) * rpc
                wait_gather(b)

                def row_body(r, _):
                    for g in range(GROUPS):
                        sl = pl.ds(g * LANES, LANES)
                        rows[b][r, sl] = rows[b][r, sl] + pos_v[h_base + r, sl]
                    return 0

                lax.fori_loop(0, rpc, row_body, 0)

                s = (nbuf // cpc) * j + b // cpc
                pltpu.async_copy(
                    rows[b],
                    out_hbm.at[pl.ds(s * seq_len + p_lo + h_base, rpc)],
                    osem[b],
                )

                kn = k + prefetch
                bn = (b + prefetch) % nbuf

                @pl.when(kn < n_chunks)
                def _prefetch():
                    @pl.when(k >= nbuf - prefetch)
                    def _drain():
                        wait_out(bn)

                    start_gather(bn, kn)

            return 0

        lax.fori_loop(0, n_chunks // nbuf, body, 0)

        for b in range(nbuf):
            wait_out(b)

    return emb


def kernel(x, tok_table, pos_table):
    B, T = x.shape
    emb = _build(B, T, 32, 4, 2)
    out = emb(x.reshape(B * T).astype(jnp.int32), tok_table, pos_table)
    return out.reshape(B, T, D_MODEL)


# speedup vs baseline: 1.4840x; 1.0112x over previous
import functools

import jax
import jax.numpy as jnp
from jax import lax
from jax.experimental import pallas as pl
from jax.experimental.pallas import tpu as pltpu
from jax.experimental.pallas import tpu_sc as plsc

D_MODEL = 768
NUM_CORES = 2
NUM_SUBCORES = 16
NW = NUM_CORES * NUM_SUBCORES
LANES = 16
GROUPS = D_MODEL // LANES

_mesh = plsc.VectorSubcoreMesh(core_axis_name="c", subcore_axis_name="s")


@functools.cache
def _build(n_seq: int, seq_len: int, rpc: int, nbuf: int, prefetch: int):
    ppw = seq_len // NW
    cpc = ppw // rpc
    n_chunks = n_seq * cpc
    total = n_seq * seq_len
    assert nbuf % cpc == 0 and n_chunks % nbuf == 0

    @functools.partial(
        pl.kernel,
        mesh=_mesh,
        out_type=jax.ShapeDtypeStruct((total, D_MODEL), jnp.float32),
        scratch_types=[
            pltpu.VMEM((n_seq * ppw,), jnp.int32),
            pltpu.VMEM((ppw, D_MODEL), jnp.float32),
        ]
        + [pltpu.VMEM((rpc, D_MODEL), jnp.float32) for _ in range(nbuf)]
        + [pltpu.SemaphoreType.DMA for _ in range(2 * nbuf)],
    )
    def emb(x_hbm, tok_hbm, pos_hbm, out_hbm, idx_all, pos_v, *bufs_sems):
        rows = bufs_sems[:nbuf]
        gsem = bufs_sems[nbuf:2 * nbuf]
        osem = bufs_sems[2 * nbuf:]
        wid = lax.axis_index("s") * NUM_CORES + lax.axis_index("c")
        p_lo = pl.multiple_of(wid * ppw, ppw)

        def idx_body(s, _):
            pltpu.async_copy(
                x_hbm.at[pl.ds(s * seq_len + p_lo, ppw)],
                idx_all.at[pl.ds(s * ppw, ppw)],
                gsem[0],
            )
            return 0

        lax.fori_loop(0, n_seq, idx_body, 0)
        pltpu.sync_copy(pos_hbm.at[pl.ds(p_lo, ppw)], pos_v)
        pltpu.make_async_copy(x_hbm.at[pl.ds(0, n_seq * ppw)],
                              idx_all, gsem[0]).wait()

        def start_gather(b, k):
            pltpu.async_copy(
                tok_hbm.at[idx_all.at[pl.ds(k * rpc, rpc)]], rows[b], gsem[b]
            )

        def wait_gather(b):
            pltpu.make_async_copy(
                tok_hbm.at[idx_all.at[pl.ds(0, rpc)]], rows[b], gsem[b]
            ).wait()

        def start_out(b, k):
            s = k // cpc
            h_off = (k % cpc) * rpc if isinstance(k, int) else lax.rem(k, cpc) * rpc
            pltpu.async_copy(
                rows[b], out_hbm.at[pl.ds(s * seq_len + p_lo + h_off, rpc)], osem[b]
            )

        def wait_out(b):
            pltpu.make_async_copy(rows[b], out_hbm.at[pl.ds(0, rpc)], osem[b]).wait()

        for k0 in range(prefetch):
            start_gather(k0, k0)

        def body(j, _):
            for b in range(nbuf):
                k = nbuf * j + b
                h_base = (b % cpc) * rpc
                wait_gather(b)

                def row_body(r, _):
                    for g in range(GROUPS):
                        sl = pl.ds(g * LANES, LANES)
                        rows[b][r, sl] = rows[b][r, sl] + pos_v[h_base + r, sl]
                    return 0

                lax.fori_loop(0, rpc, row_body, 0)

                s = (nbuf // cpc) * j + b // cpc
                pltpu.async_copy(
                    rows[b],
                    out_hbm.at[pl.ds(s * seq_len + p_lo + h_base, rpc)],
                    osem[b],
                )

                kn = k + prefetch
                bn = (b + prefetch) % nbuf

                @pl.when(kn < n_chunks)
                def _prefetch():
                    @pl.when(k >= nbuf - prefetch)
                    def _drain():
                        wait_out(bn)

                    start_gather(bn, kn)

            return 0

        lax.fori_loop(0, n_chunks // nbuf, body, 0)

        for b in range(nbuf):
            wait_out(b)

    return emb


def kernel(x, tok_table, pos_table):
    B, T = x.shape
    emb = _build(B, T, 32, 4, 3)
    out = emb(x.reshape(B * T).astype(jnp.int32), tok_table, pos_table)
    return out.reshape(B, T, D_MODEL)


# speedup vs baseline: 1.4896x; 1.0038x over previous
import functools

import jax
import jax.numpy as jnp
from jax import lax
from jax.experimental import pallas as pl
from jax.experimental.pallas import tpu as pltpu
from jax.experimental.pallas import tpu_sc as plsc

D_MODEL = 768
NUM_CORES = 2
NUM_SUBCORES = 16
NW = NUM_CORES * NUM_SUBCORES
LANES = 16
GROUPS = D_MODEL // LANES

_mesh = plsc.VectorSubcoreMesh(core_axis_name="c", subcore_axis_name="s")


@functools.cache
def _build(n_seq: int, seq_len: int, rpc: int, nbuf: int, prefetch: int):
    ppw = seq_len // NW
    cpc = ppw // rpc
    n_chunks = n_seq * cpc
    total = n_seq * seq_len
    assert nbuf % cpc == 0 and n_chunks % nbuf == 0

    @functools.partial(
        pl.kernel,
        mesh=_mesh,
        out_type=jax.ShapeDtypeStruct((total, D_MODEL), jnp.float32),
        scratch_types=[
            pltpu.VMEM((n_seq * ppw,), jnp.int32),
            pltpu.VMEM((ppw, D_MODEL), jnp.float32),
        ]
        + [pltpu.VMEM((rpc, D_MODEL), jnp.float32) for _ in range(nbuf)]
        + [pltpu.SemaphoreType.DMA for _ in range(2 * nbuf + 1)],
    )
    def emb(x_hbm, tok_hbm, pos_hbm, out_hbm, idx_all, pos_v, *bufs_sems):
        rows = bufs_sems[:nbuf]
        gsem = bufs_sems[nbuf:2 * nbuf]
        osem = bufs_sems[2 * nbuf:3 * nbuf]
        isem = bufs_sems[3 * nbuf]
        wid = lax.axis_index("s") * NUM_CORES + lax.axis_index("c")
        p_lo = pl.multiple_of(wid * ppw, ppw)

        def idx_copy(s):
            pltpu.async_copy(
                x_hbm.at[pl.ds(s * seq_len + p_lo, ppw)],
                idx_all.at[pl.ds(s * ppw, ppw)],
                isem,
            )

        n_first = (prefetch * rpc + ppw - 1) // ppw
        for s0 in range(n_first):
            idx_copy(s0)
        pltpu.make_async_copy(x_hbm.at[pl.ds(0, n_first * ppw)],
                              idx_all.at[pl.ds(0, n_first * ppw)],
                              isem).wait()

        def start_gather(b, k):
            pltpu.async_copy(
                tok_hbm.at[idx_all.at[pl.ds(k * rpc, rpc)]], rows[b], gsem[b]
            )

        def wait_gather(b):
            pltpu.make_async_copy(
                tok_hbm.at[idx_all.at[pl.ds(0, rpc)]], rows[b], gsem[b]
            ).wait()

        def start_out(b, k):
            s = k // cpc
            h_off = (k % cpc) * rpc if isinstance(k, int) else lax.rem(k, cpc) * rpc
            pltpu.async_copy(
                rows[b], out_hbm.at[pl.ds(s * seq_len + p_lo + h_off, rpc)], osem[b]
            )

        def wait_out(b):
            pltpu.make_async_copy(rows[b], out_hbm.at[pl.ds(0, rpc)], osem[b]).wait()

        for k0 in range(prefetch):
            start_gather(k0, k0)

        def idx_body(s, _):
            idx_copy(s)
            return 0

        lax.fori_loop(n_first, n_seq, idx_body, 0)
        pltpu.sync_copy(pos_hbm.at[pl.ds(p_lo, ppw)], pos_v)
        pltpu.make_async_copy(
            x_hbm.at[pl.ds(0, (n_seq - n_first) * ppw)],
            idx_all.at[pl.ds(0, (n_seq - n_first) * ppw)],
            isem,
        ).wait()

        def body(j, _):
            for b in range(nbuf):
                k = nbuf * j + b
                h_base = (b % cpc) * rpc
                wait_gather(b)

                def row_body(r, _):
                    for g in range(GROUPS):
                        sl = pl.ds(g * LANES, LANES)
                        rows[b][r, sl] = rows[b][r, sl] + pos_v[h_base + r, sl]
                    return 0

                lax.fori_loop(0, rpc, row_body, 0)

                s = (nbuf // cpc) * j + b // cpc
                pltpu.async_copy(
                    rows[b],
                    out_hbm.at[pl.ds(s * seq_len + p_lo + h_base, rpc)],
                    osem[b],
                )

                kn = k + prefetch
                bn = (b + prefetch) % nbuf

                @pl.when(kn < n_chunks)
                def _prefetch():
                    @pl.when(k >= nbuf - prefetch)
                    def _drain():
                        wait_out(bn)

                    start_gather(bn, kn)

            return 0

        lax.fori_loop(0, n_chunks // nbuf, body, 0)

        for b in range(nbuf):
            wait_out(b)

    return emb


def kernel(x, tok_table, pos_table):
    B, T = x.shape
    emb = _build(B, T, 32, 4, 3)
    out = emb(x.reshape(B * T).astype(jnp.int32), tok_table, pos_table)
    return out.reshape(B, T, D_MODEL)


# speedup vs baseline: 1.4942x; 1.0031x over previous
import functools

import jax
import jax.numpy as jnp
from jax import lax
from jax.experimental import pallas as pl
from jax.experimental.pallas import tpu as pltpu
from jax.experimental.pallas import tpu_sc as plsc

D_MODEL = 768
NUM_CORES = 2
NUM_SUBCORES = 16
NW = NUM_CORES * NUM_SUBCORES
LANES = 16
GROUPS = D_MODEL // LANES

_mesh = plsc.VectorSubcoreMesh(core_axis_name="c", subcore_axis_name="s")


@functools.cache
def _build(n_seq: int, seq_len: int, rpc: int, nbuf: int, prefetch: int):
    ppw = seq_len // NW
    cpc = ppw // rpc
    n_chunks = n_seq * cpc
    total = n_seq * seq_len
    assert nbuf % cpc == 0 and n_chunks % nbuf == 0

    @functools.partial(
        pl.kernel,
        mesh=_mesh,
        out_type=jax.ShapeDtypeStruct((total, D_MODEL), jnp.float32),
        scratch_types=[
            pltpu.VMEM((n_seq * ppw,), jnp.int32),
            pltpu.VMEM((ppw, D_MODEL), jnp.float32),
        ]
        + [pltpu.VMEM((rpc, D_MODEL), jnp.float32) for _ in range(nbuf)]
        + [pltpu.SemaphoreType.DMA for _ in range(2 * nbuf + 1)],
    )
    def emb(x_hbm, tok_hbm, pos_hbm, out_hbm, idx_all, pos_v, *bufs_sems):
        rows = bufs_sems[:nbuf]
        gsem = bufs_sems[nbuf:2 * nbuf]
        osem = bufs_sems[2 * nbuf:3 * nbuf]
        isem = bufs_sems[3 * nbuf]
        wid = lax.axis_index("s") * NUM_CORES + lax.axis_index("c")
        p_lo = pl.multiple_of(wid * ppw, ppw)

        def idx_copy(s):
            pltpu.async_copy(
                x_hbm.at[pl.ds(s * seq_len + p_lo, ppw)],
                idx_all.at[pl.ds(s * ppw, ppw)],
                isem,
            )

        n_first = (prefetch * rpc + ppw - 1) // ppw
        for s0 in range(n_first):
            idx_copy(s0)
        pltpu.make_async_copy(x_hbm.at[pl.ds(0, n_first * ppw)],
                              idx_all.at[pl.ds(0, n_first * ppw)],
                              isem).wait()

        def start_gather(b, k):
            pltpu.async_copy(
                tok_hbm.at[idx_all.at[pl.ds(k * rpc, rpc)]], rows[b], gsem[b]
            )

        def wait_gather(b):
            pltpu.make_async_copy(
                tok_hbm.at[idx_all.at[pl.ds(0, rpc)]], rows[b], gsem[b]
            ).wait()

        def wait_out(b):
            pltpu.make_async_copy(rows[b], out_hbm.at[pl.ds(0, rpc)], osem[b]).wait()

        for k0 in range(prefetch):
            start_gather(k0, k0)

        def idx_body(s, _):
            idx_copy(s)
            return 0

        lax.fori_loop(n_first, n_seq, idx_body, 0)
        pltpu.sync_copy(pos_hbm.at[pl.ds(p_lo, ppw)], pos_v)
        pltpu.make_async_copy(
            x_hbm.at[pl.ds(0, (n_seq - n_first) * ppw)],
            idx_all.at[pl.ds(0, (n_seq - n_first) * ppw)],
            isem,
        ).wait()

        def body(j, _):
            for b in range(nbuf):
                k = nbuf * j + b
                h_base = (b % cpc) * rpc
                wait_gather(b)

                def row_body(r, _):
                    for g in range(GROUPS):
                        sl = pl.ds(g * LANES, LANES)
                        rows[b][r, sl] = rows[b][r, sl] + pos_v[h_base + r, sl]
                    return 0

                lax.fori_loop(0, rpc, row_body, 0)

                s = (nbuf // cpc) * j + b // cpc
                pltpu.async_copy(
                    rows[b],
                    out_hbm.at[pl.ds(s * seq_len + p_lo + h_base, rpc)],
                    osem[b],
                )

                kn = k + prefetch
                bn = (b + prefetch) % nbuf

                @pl.when(kn < n_chunks)
                def _prefetch():
                    @pl.when(k >= nbuf - prefetch)
                    def _drain():
                        wait_out(bn)

                    start_gather(bn, kn)

            return 0

        lax.fori_loop(0, n_chunks // nbuf, body, 0)

        for b in range(nbuf):
            wait_out(b)

    return emb


def kernel(x, tok_table, pos_table):
    B, T = x.shape
    emb = _build(B, T, 32, 4, 3)
    out = emb(x.reshape(B * T).astype(jnp.int32), tok_table, pos_table)
    return out.reshape(B, T, D_MODEL)
